# baseline (speedup 1.0000x reference)
import functools

import jax
import jax.numpy as jnp
from jax import lax
from jax.experimental import pallas as pl
from jax.experimental.pallas import tpu as pltpu
from jax.experimental.pallas import tpu_sc as plsc

NC, NS = 2, 16
NW = NC * NS
CH = 128

_mesh = lambda: plsc.VectorSubcoreMesh(core_axis_name="c", subcore_axis_name="s")


def _row_partition(n):
    rpt = (n // NS) & ~7
    tail = n - rpt * NS
    assert tail % 8 == 0
    return rpt, tail


def _copy_rows(s, src, dst, rpt, tail, add=False):
    pltpu.sync_copy(src.at[pl.ds(s * rpt, rpt)],
                    dst.at[pl.ds(s * rpt, rpt)], add=add)
    if tail:
        @pl.when(s == NS - 1)
        def _():
            pltpu.sync_copy(src.at[pl.ds(NS * rpt, tail)],
                            dst.at[pl.ds(NS * rpt, tail)], add=add)


def _sc_degree(ed2d, ones, zeros, n, d):
    nchunk, rpw = _chunk_partition(ed2d.shape[0] * CH)
    rpt, tail = _row_partition(n)

    @functools.partial(
        pl.kernel, mesh=_mesh(),
        out_type=jax.ShapeDtypeStruct((NC, n, d), jnp.float32),
        scratch_types=[
            pltpu.VMEM((rpw, CH), jnp.int32),
            pltpu.VMEM((CH, d), jnp.float32),
            pltpu.VMEM_SHARED((n, d), jnp.float32),
        ],
    )
    def k(edst_hbm, ones_hbm, zeros_hbm, out_hbm, idv, ones_v, acc):
        c = lax.axis_index("c")
        s = lax.axis_index("s")
        wid = s * NC + c
        _copy_rows(s, zeros_hbm, acc, rpt, tail)
        pltpu.sync_copy(ones_hbm, ones_v)
        plsc.subcore_barrier()
        row0 = wid * rpw
        pltpu.sync_copy(edst_hbm.at[pl.ds(row0, rpw)], idv)

        def step(j, carry):
            pltpu.sync_copy(ones_v, acc.at[idv.at[j]], add=True)
            return carry

        lax.fori_loop(0, rpw, step, 0)
        plsc.subcore_barrier()
        _copy_rows(s, acc, out_hbm.at[c], rpt, tail)

    return k(ed2d, ones, zeros)


def _chunk_partition(E):
    nchunk = E // CH
    assert E % CH == 0 and nchunk % NW == 0
    rpw = nchunk // NW
    assert rpw % 8 == 0
    return nchunk, rpw


def _sc_aggregate(table, es2d, ed2d, zeros, n, d):
    nchunk, rpw = _chunk_partition(es2d.shape[0] * CH)
    rpt, tail = _row_partition(n)
    IP = 16
    assert rpw % IP == 0 and IP % 8 == 0

    @functools.partial(
        pl.kernel, mesh=_mesh(),
        out_type=jax.ShapeDtypeStruct((NC, n, d), jnp.float32),
        scratch_types=[
            pltpu.VMEM((IP, CH), jnp.int32),
            pltpu.VMEM((IP, CH), jnp.int32),
            pltpu.VMEM((CH, d), jnp.float32),
            pltpu.VMEM((CH, d), jnp.float32),
            pltpu.SemaphoreType.DMA,
            pltpu.SemaphoreType.DMA,
            pltpu.VMEM_SHARED((n, d), jnp.float32),
        ],
    )
    def k(tab_hbm, esrc_hbm, edst_hbm, zeros_hbm, out_hbm,
          isv, idv, rows0, rows1, sem0, sem1, acc):
        c = lax.axis_index("c")
        s = lax.axis_index("s")
        wid = s * NC + c
        _copy_rows(s, zeros_hbm, acc, rpt, tail)
        plsc.subcore_barrier()
        row0 = wid * rpw

        for p in range(rpw // IP):
            pltpu.sync_copy(esrc_hbm.at[pl.ds(row0 + p * IP, IP)], isv)
            pltpu.sync_copy(edst_hbm.at[pl.ds(row0 + p * IP, IP)], idv)
            pltpu.async_copy(tab_hbm.at[isv.at[0]], rows0, sem0)

            def pair(g, carry):
                j0 = 2 * g
                pltpu.make_async_copy(tab_hbm.at[isv.at[j0]],
                                      rows0, sem0).wait()
                pltpu.async_copy(tab_hbm.at[isv.at[j0 + 1]], rows1, sem1)
                pltpu.sync_copy(rows0, acc.at[idv.at[j0]], add=True)
                pltpu.make_async_copy(tab_hbm.at[isv.at[j0 + 1]],
                                      rows1, sem1).wait()

                @pl.when(j0 + 2 < IP)
                def _():
                    pltpu.async_copy(tab_hbm.at[isv.at[j0 + 2]], rows0, sem0)

                pltpu.sync_copy(rows1, acc.at[idv.at[j0 + 1]], add=True)
                return carry

            lax.fori_loop(0, IP // 2, pair, 0)
        plsc.subcore_barrier()
        _copy_rows(s, acc, out_hbm.at[c], rpt, tail)

    return k(table, es2d, ed2d, zeros)


def _sc_edge_gather(a0, a1, es2d, ed2d, d):
    nchunk, rpw = _chunk_partition(es2d.shape[0] * CH)
    assert d % 16 == 0
    dl = d // 16

    @functools.partial(
        pl.kernel, mesh=_mesh(),
        out_type=jax.ShapeDtypeStruct((nchunk, CH, d), jnp.float32),
        scratch_types=[
            pltpu.VMEM((rpw, CH), jnp.int32),
            pltpu.VMEM((rpw, CH), jnp.int32),
            pltpu.VMEM((CH, d), jnp.float32),
            pltpu.VMEM((CH, d), jnp.float32),
            pltpu.VMEM((CH, d), jnp.float32),
            pltpu.VMEM((CH, d), jnp.float32),
            pltpu.SemaphoreType.DMA,
            pltpu.SemaphoreType.DMA,
            pltpu.SemaphoreType.DMA,
            pltpu.SemaphoreType.DMA,
        ],
    )
    def k(a0_hbm, a1_hbm, esrc_hbm, edst_hbm, out_hbm,
          isv, idv, ra0, rb0, ra1, rb1, sa0, sb0, sa1, sb1):
        c = lax.axis_index("c")
        s = lax.axis_index("s")
        wid = s * NC + c
        row0 = wid * rpw

        def add_rows(ra, rb):
            def add_row(r, carry):
                for kk in range(dl):
                    col = kk * 16
                    ra[r, pl.ds(col, 16)] = (ra[r, pl.ds(col, 16)]
                                             + rb[r, pl.ds(col, 16)])
                return carry

            lax.fori_loop(0, CH, add_row, 0)

        def gathers(j, ra, rb, sa, sb):
            pltpu.async_copy(a0_hbm.at[isv.at[j]], ra, sa)
            pltpu.async_copy(a1_hbm.at[idv.at[j]], rb, sb)

        def waits(j, ra, rb, sa, sb):
            pltpu.make_async_copy(a0_hbm.at[isv.at[j]], ra, sa).wait()
            pltpu.make_async_copy(a1_hbm.at[idv.at[j]], rb, sb).wait()

        pltpu.sync_copy(esrc_hbm.at[pl.ds(row0, rpw)], isv)
        pltpu.sync_copy(edst_hbm.at[pl.ds(row0, rpw)], idv)
        gathers(0, ra0, rb0, sa0, sb0)
        gathers(1, ra1, rb1, sa1, sb1)

        def pair(g, carry):
            j0 = 2 * g
            j1 = j0 + 1
            waits(j0, ra0, rb0, sa0, sb0)
            add_rows(ra0, rb0)
            pltpu.sync_copy(ra0, out_hbm.at[row0 + j0])

            @pl.when(j0 + 2 < rpw)
            def _():
                gathers(j0 + 2, ra0, rb0, sa0, sb0)

            waits(j1, ra1, rb1, sa1, sb1)
            add_rows(ra1, rb1)
            pltpu.sync_copy(ra1, out_hbm.at[row0 + j1])

            @pl.when(j1 + 2 < rpw)
            def _():
                gathers(j1 + 2, ra1, rb1, sa1, sb1)

            return carry

        lax.fori_loop(0, rpw // 2, pair, 0)

    return k(a0, a1, es2d, ed2d)


def _tc_prescale(d0, d1, x, n_real):
    n, d = x.shape
    R = 1000
    assert n_real % R == 0

    def body(d0r, d1r, xr, xs_o, dinv_o):
        deg = jnp.maximum(d0r[:, 0:1] + d1r[:, 0:1] + 1.0, 1.0)
        dv = lax.rsqrt(deg)
        dinv_o[...] = dv
        xs_o[...] = xr[...] * dv

    return pl.pallas_call(
        body,
        grid=(n_real // R,),
        in_specs=[pl.BlockSpec((R, d), lambda i: (i, 0)),
                  pl.BlockSpec((R, d), lambda i: (i, 0)),
                  pl.BlockSpec((R, d), lambda i: (i, 0))],
        out_specs=[pl.BlockSpec((R, d), lambda i: (i, 0)),
                   pl.BlockSpec((R, 1), lambda i: (i, 0))],
        out_shape=[jax.ShapeDtypeStruct((n, d), jnp.float32),
                   jax.ShapeDtypeStruct((n, 1), jnp.float32)],
    )(d0, d1, x)


def _tc_mid(a0, a1, xs, dinv, W1, b1, W2, n_real):
    n, d = xs.shape
    h = W1.shape[1]
    R = 1000
    assert n_real % R == 0

    def body(a0r, a1r, xsr, dvr, w1r, b1r, w2r, ys_o):
        dv = dvr[...]
        px = dv * (a0r[...] + a1r[...] + xsr[...])
        z1 = jnp.maximum(
            jnp.dot(px, w1r[...], preferred_element_type=jnp.float32)
            + b1r[...], 0.0)
        y1 = jnp.dot(z1, w2r[...], preferred_element_type=jnp.float32)
        ys_o[...] = y1 * dv

    return pl.pallas_call(
        body,
        grid=(n_real // R,),
        in_specs=[pl.BlockSpec((R, d), lambda i: (i, 0)),
                  pl.BlockSpec((R, d), lambda i: (i, 0)),
                  pl.BlockSpec((R, d), lambda i: (i, 0)),
                  pl.BlockSpec((R, 1), lambda i: (i, 0)),
                  pl.BlockSpec((d, h), lambda i: (0, 0)),
                  pl.BlockSpec((1, h), lambda i: (0, 0)),
                  pl.BlockSpec((h, d), lambda i: (0, 0))],
        out_specs=pl.BlockSpec((R, d), lambda i: (i, 0)),
        out_shape=jax.ShapeDtypeStruct((n, d), jnp.float32),
    )(a0, a1, xs, dinv, W1, b1, W2)


def _tc_decoder_pre(a0, a1, ys, dinv, b2, fc1_t, fc1_b_half, fc1_bias, n_real):
    n, d = ys.shape
    R = 1000
    assert n_real % R == 0

    def body(a0r, a1r, ysr, dvr, b2r, wtr, wbr, fbr, o0, o1):
        z2 = dvr[...] * (a0r[...] + a1r[...] + ysr[...]) + b2r[...]
        o0[...] = jnp.dot(z2, wtr[...],
                          preferred_element_type=jnp.float32) + fbr[...]
        o1[...] = jnp.dot(z2, wbr[...], preferred_element_type=jnp.float32)

    return pl.pallas_call(
        body,
        grid=(n_real // R,),
        in_specs=[pl.BlockSpec((R, d), lambda i: (i, 0)),
                  pl.BlockSpec((R, d), lambda i: (i, 0)),
                  pl.BlockSpec((R, d), lambda i: (i, 0)),
                  pl.BlockSpec((R, 1), lambda i: (i, 0)),
                  pl.BlockSpec((1, d), lambda i: (0, 0)),
                  pl.BlockSpec((d, d), lambda i: (0, 0)),
                  pl.BlockSpec((d, d), lambda i: (0, 0)),
                  pl.BlockSpec((1, d), lambda i: (0, 0))],
        out_specs=[pl.BlockSpec((R, d), lambda i: (i, 0)),
                   pl.BlockSpec((R, d), lambda i: (i, 0))],
        out_shape=[jax.ShapeDtypeStruct((n, d), jnp.float32),
                   jax.ShapeDtypeStruct((n, d), jnp.float32)],
    )(a0, a1, ys, dinv, b2, fc1_t, fc1_b_half, fc1_bias)


def _tc_decoder_mlp(g, f2, b2, f3, b3, f4, b4):
    E, d = g.shape
    h2, h3 = f2.shape[1], f3.shape[1]
    EB = 2048
    assert E % EB == 0

    def body(gr, f2r, b2r, f3r, b3r, f4r, b4r, o):
        t = jnp.maximum(gr[...], 0.0)
        t = jnp.maximum(
            jnp.dot(t, f2r[...], preferred_element_type=jnp.float32)
            + b2r[...], 0.0)
        t = jnp.maximum(
            jnp.dot(t, f3r[...], preferred_element_type=jnp.float32)
            + b3r[...], 0.0)
        o[...] = jnp.dot(t, f4r[...],
                         preferred_element_type=jnp.float32) + b4r[...]

    return pl.pallas_call(
        body,
        grid=(E // EB,),
        in_specs=[pl.BlockSpec((EB, d), lambda i: (i, 0)),
                  pl.BlockSpec((d, h2), lambda i: (0, 0)),
                  pl.BlockSpec((1, h2), lambda i: (0, 0)),
                  pl.BlockSpec((h2, h3), lambda i: (0, 0)),
                  pl.BlockSpec((1, h3), lambda i: (0, 0)),
                  pl.BlockSpec((h3, 1), lambda i: (0, 0)),
                  pl.BlockSpec((1, 1), lambda i: (0, 0))],
        out_specs=pl.BlockSpec((EB, 1), lambda i: (i, 0)),
        out_shape=jax.ShapeDtypeStruct((E, 1), jnp.float32),
    )(g, f2, b2, f3, b3, f4, b4)


def kernel(x, edge_index, W1, b1, W2, b2, fc1_W, fc1_b,
           fc2_W, fc2_b, fc3_W, fc3_b, fc4_W, fc4_b):
    n, d = x.shape
    E = edge_index.shape[1]
    nchunk = -(-E // CH)
    rpw = (-(-nchunk // NW) + 7) & ~7
    epad = rpw * NW * CH
    npd = n + 8
    ei = edge_index.astype(jnp.int32)
    fill = jnp.full((epad - E,), n, jnp.int32)
    es2d = jnp.concatenate([ei[0], fill]).reshape(epad // CH, CH)
    ed2d = jnp.concatenate([ei[1], fill]).reshape(epad // CH, CH)
    x_pad = jnp.concatenate([x, jnp.zeros((npd - n, d), jnp.float32)])

    zerosd = jnp.zeros((npd, d), jnp.float32)
    ones = jnp.ones((CH, d), jnp.float32)

    deg = _sc_degree(ed2d, ones, zerosd, npd, d)
    xs, dinv = _tc_prescale(deg[0], deg[1], x_pad, n)
    agg1 = _sc_aggregate(xs, es2d, ed2d, zerosd, npd, d)
    ys = _tc_mid(agg1[0], agg1[1], xs, dinv, W1, b1.reshape(1, -1), W2, n)
    agg2 = _sc_aggregate(ys, es2d, ed2d, zerosd, npd, d)
    a0, a1 = _tc_decoder_pre(agg2[0], agg2[1], ys, dinv, b2.reshape(1, -1),
                             fc1_W[:d], fc1_W[d:], fc1_b.reshape(1, -1), n)
    g = _sc_edge_gather(a0, a1, es2d, ed2d, d).reshape(epad, d)
    out = _tc_decoder_mlp(g, fc2_W, fc2_b.reshape(1, -1),
                          fc3_W, fc3_b.reshape(1, -1),
                          fc4_W, fc4_b.reshape(1, -1))
    return jnp.squeeze(out, axis=-1)[:E]


# speedup vs baseline: 2.1499x; 2.1499x over previous
import functools

import jax
import jax.numpy as jnp
from jax import lax
from jax.experimental import pallas as pl
from jax.experimental.pallas import tpu as pltpu
from jax.experimental.pallas import tpu_sc as plsc

NC, NS = 2, 16
NW = NC * NS
CH = 128

_mesh = lambda: plsc.VectorSubcoreMesh(core_axis_name="c", subcore_axis_name="s")


def _row_partition(n):
    rpt = (n // NS) & ~7
    tail = n - rpt * NS
    assert tail % 8 == 0
    return rpt, tail


def _copy_rows(s, src, dst, rpt, tail, add=False):
    pltpu.sync_copy(src.at[pl.ds(s * rpt, rpt)],
                    dst.at[pl.ds(s * rpt, rpt)], add=add)
    if tail:
        @pl.when(s == NS - 1)
        def _():
            pltpu.sync_copy(src.at[pl.ds(NS * rpt, tail)],
                            dst.at[pl.ds(NS * rpt, tail)], add=add)


def _sc_degree(ed2d, ones, zeros, n, d):
    nchunk, rpw = _chunk_partition(ed2d.shape[0] * CH)
    rpt, tail = _row_partition(n)

    @functools.partial(
        pl.kernel, mesh=_mesh(),
        out_type=jax.ShapeDtypeStruct((NC, n, d), jnp.float32),
        scratch_types=[
            pltpu.VMEM((rpw, CH), jnp.int32),
            pltpu.VMEM((CH, d), jnp.float32),
            pltpu.VMEM_SHARED((n, d), jnp.float32),
        ],
    )
    def k(edst_hbm, ones_hbm, zeros_hbm, out_hbm, idv, ones_v, acc):
        c = lax.axis_index("c")
        s = lax.axis_index("s")
        wid = s * NC + c
        _copy_rows(s, zeros_hbm, acc, rpt, tail)
        pltpu.sync_copy(ones_hbm, ones_v)
        plsc.subcore_barrier()
        row0 = wid * rpw
        pltpu.sync_copy(edst_hbm.at[pl.ds(row0, rpw)], idv)

        def step(j, carry):
            pltpu.sync_copy(ones_v, acc.at[idv.at[j]], add=True)
            return carry

        lax.fori_loop(0, rpw, step, 0)
        plsc.subcore_barrier()
        _copy_rows(s, acc, out_hbm.at[c], rpt, tail)

    return k(ed2d, ones, zeros)


def _chunk_partition(E):
    nchunk = E // CH
    assert E % CH == 0 and nchunk % NW == 0
    rpw = nchunk // NW
    assert rpw % 8 == 0
    return nchunk, rpw


def _sc_aggregate(table, es2d, ed2d, zeros, n, d):
    nchunk, rpw = _chunk_partition(es2d.shape[0] * CH)
    rpt, tail = _row_partition(n)
    IP = 16
    assert rpw % IP == 0 and IP % 8 == 0

    @functools.partial(
        pl.kernel, mesh=_mesh(),
        out_type=jax.ShapeDtypeStruct((NC, n, d), jnp.float32),
        scratch_types=[
            pltpu.VMEM((IP, CH), jnp.int32),
            pltpu.VMEM((IP, CH), jnp.int32),
            pltpu.VMEM((CH, d), jnp.float32),
            pltpu.VMEM((CH, d), jnp.float32),
            pltpu.SemaphoreType.DMA,
            pltpu.SemaphoreType.DMA,
            pltpu.VMEM_SHARED((n, d), jnp.float32),
        ],
    )
    def k(tab_hbm, esrc_hbm, edst_hbm, zeros_hbm, out_hbm,
          isv, idv, rows0, rows1, sem0, sem1, acc):
        c = lax.axis_index("c")
        s = lax.axis_index("s")
        wid = s * NC + c
        _copy_rows(s, zeros_hbm, acc, rpt, tail)
        plsc.subcore_barrier()
        row0 = wid * rpw

        for p in range(rpw // IP):
            pltpu.sync_copy(esrc_hbm.at[pl.ds(row0 + p * IP, IP)], isv)
            pltpu.sync_copy(edst_hbm.at[pl.ds(row0 + p * IP, IP)], idv)
            pltpu.async_copy(tab_hbm.at[isv.at[0]], rows0, sem0)

            def pair(g, carry):
                j0 = 2 * g
                pltpu.make_async_copy(tab_hbm.at[isv.at[j0]],
                                      rows0, sem0).wait()
                pltpu.async_copy(tab_hbm.at[isv.at[j0 + 1]], rows1, sem1)
                pltpu.sync_copy(rows0, acc.at[idv.at[j0]], add=True)
                pltpu.make_async_copy(tab_hbm.at[isv.at[j0 + 1]],
                                      rows1, sem1).wait()

                @pl.when(j0 + 2 < IP)
                def _():
                    pltpu.async_copy(tab_hbm.at[isv.at[j0 + 2]], rows0, sem0)

                pltpu.sync_copy(rows1, acc.at[idv.at[j0 + 1]], add=True)
                return carry

            lax.fori_loop(0, IP // 2, pair, 0)
        plsc.subcore_barrier()
        _copy_rows(s, acc, out_hbm.at[c], rpt, tail)

    return k(table, es2d, ed2d, zeros)


def _sc_edge_gather(a0, a1, es2d, ed2d, d):
    nchunk, rpw = _chunk_partition(es2d.shape[0] * CH)
    assert d % 16 == 0
    dl = d // 16

    @functools.partial(
        pl.kernel, mesh=_mesh(),
        out_type=jax.ShapeDtypeStruct((nchunk, CH, d), jnp.float32),
        scratch_types=[
            pltpu.VMEM((rpw, CH), jnp.int32),
            pltpu.VMEM((rpw, CH), jnp.int32),
            pltpu.VMEM((CH, d), jnp.float32),
            pltpu.VMEM((CH, d), jnp.float32),
            pltpu.VMEM((CH, d), jnp.float32),
            pltpu.VMEM((CH, d), jnp.float32),
            pltpu.SemaphoreType.DMA,
            pltpu.SemaphoreType.DMA,
            pltpu.SemaphoreType.DMA,
            pltpu.SemaphoreType.DMA,
        ],
    )
    def k(a0_hbm, a1_hbm, esrc_hbm, edst_hbm, out_hbm,
          isv, idv, ra0, rb0, ra1, rb1, sa0, sb0, sa1, sb1):
        c = lax.axis_index("c")
        s = lax.axis_index("s")
        wid = s * NC + c
        row0 = wid * rpw

        def add_rows(ra, rb):
            def add_row(r, carry):
                for kk in range(dl):
                    col = kk * 16
                    ra[r, pl.ds(col, 16)] = (ra[r, pl.ds(col, 16)]
                                             + rb[r, pl.ds(col, 16)])
                return carry

            lax.fori_loop(0, CH, add_row, 0)

        def gathers(j, ra, rb, sa, sb):
            pltpu.async_copy(a0_hbm.at[isv.at[j]], ra, sa)
            pltpu.async_copy(a1_hbm.at[idv.at[j]], rb, sb)

        def waits(j, ra, rb, sa, sb):
            pltpu.make_async_copy(a0_hbm.at[isv.at[j]], ra, sa).wait()
            pltpu.make_async_copy(a1_hbm.at[idv.at[j]], rb, sb).wait()

        pltpu.sync_copy(esrc_hbm.at[pl.ds(row0, rpw)], isv)
        pltpu.sync_copy(edst_hbm.at[pl.ds(row0, rpw)], idv)
        gathers(0, ra0, rb0, sa0, sb0)
        gathers(1, ra1, rb1, sa1, sb1)

        def pair(g, carry):
            j0 = 2 * g
            j1 = j0 + 1
            waits(j0, ra0, rb0, sa0, sb0)
            add_rows(ra0, rb0)
            pltpu.sync_copy(ra0, out_hbm.at[row0 + j0])

            @pl.when(j0 + 2 < rpw)
            def _():
                gathers(j0 + 2, ra0, rb0, sa0, sb0)

            waits(j1, ra1, rb1, sa1, sb1)
            add_rows(ra1, rb1)
            pltpu.sync_copy(ra1, out_hbm.at[row0 + j1])

            @pl.when(j1 + 2 < rpw)
            def _():
                gathers(j1 + 2, ra1, rb1, sa1, sb1)

            return carry

        lax.fori_loop(0, rpw // 2, pair, 0)

    return k(a0, a1, es2d, ed2d)


def _tc_prescale(d0, d1, x, n_real):
    n, d = x.shape
    R = 1000
    assert n_real % R == 0

    def body(d0r, d1r, xr, xs_o, dinv_o):
        deg = jnp.maximum(d0r[:, 0:1] + d1r[:, 0:1] + 1.0, 1.0)
        dv = lax.rsqrt(deg)
        dinv_o[...] = dv
        xs_o[...] = xr[...] * dv

    return pl.pallas_call(
        body,
        grid=(n_real // R,),
        in_specs=[pl.BlockSpec((R, d), lambda i: (i, 0)),
                  pl.BlockSpec((R, d), lambda i: (i, 0)),
                  pl.BlockSpec((R, d), lambda i: (i, 0))],
        out_specs=[pl.BlockSpec((R, d), lambda i: (i, 0)),
                   pl.BlockSpec((R, 1), lambda i: (i, 0))],
        out_shape=[jax.ShapeDtypeStruct((n, d), jnp.float32),
                   jax.ShapeDtypeStruct((n, 1), jnp.float32)],
    )(d0, d1, x)


def _tc_mid(a0, a1, xs, dinv, W1, b1, W2, n_real):
    n, d = xs.shape
    h = W1.shape[1]
    R = 1000
    assert n_real % R == 0

    def body(a0r, a1r, xsr, dvr, w1r, b1r, w2r, ys_o):
        dv = dvr[...]
        px = dv * (a0r[...] + a1r[...] + xsr[...])
        z1 = jnp.maximum(
            jnp.dot(px, w1r[...], preferred_element_type=jnp.float32)
            + b1r[...], 0.0)
        y1 = jnp.dot(z1, w2r[...], preferred_element_type=jnp.float32)
        ys_o[...] = y1 * dv

    return pl.pallas_call(
        body,
        grid=(n_real // R,),
        in_specs=[pl.BlockSpec((R, d), lambda i: (i, 0)),
                  pl.BlockSpec((R, d), lambda i: (i, 0)),
                  pl.BlockSpec((R, d), lambda i: (i, 0)),
                  pl.BlockSpec((R, 1), lambda i: (i, 0)),
                  pl.BlockSpec((d, h), lambda i: (0, 0)),
                  pl.BlockSpec((1, h), lambda i: (0, 0)),
                  pl.BlockSpec((h, d), lambda i: (0, 0))],
        out_specs=pl.BlockSpec((R, d), lambda i: (i, 0)),
        out_shape=jax.ShapeDtypeStruct((n, d), jnp.float32),
    )(a0, a1, xs, dinv, W1, b1, W2)


def _tc_decoder_pre(a0, a1, ys, dinv, b2, fc1_t, fc1_b_half, fc1_bias, n_real):
    n, d = ys.shape
    R = 1000
    assert n_real % R == 0

    def body(a0r, a1r, ysr, dvr, b2r, wtr, wbr, fbr, o0, o1):
        z2 = dvr[...] * (a0r[...] + a1r[...] + ysr[...]) + b2r[...]
        o0[...] = jnp.dot(z2, wtr[...],
                          preferred_element_type=jnp.float32) + fbr[...]
        o1[...] = jnp.dot(z2, wbr[...], preferred_element_type=jnp.float32)

    return pl.pallas_call(
        body,
        grid=(n_real // R,),
        in_specs=[pl.BlockSpec((R, d), lambda i: (i, 0)),
                  pl.BlockSpec((R, d), lambda i: (i, 0)),
                  pl.BlockSpec((R, d), lambda i: (i, 0)),
                  pl.BlockSpec((R, 1), lambda i: (i, 0)),
                  pl.BlockSpec((1, d), lambda i: (0, 0)),
                  pl.BlockSpec((d, d), lambda i: (0, 0)),
                  pl.BlockSpec((d, d), lambda i: (0, 0)),
                  pl.BlockSpec((1, d), lambda i: (0, 0))],
        out_specs=[pl.BlockSpec((R, d), lambda i: (i, 0)),
                   pl.BlockSpec((R, d), lambda i: (i, 0))],
        out_shape=[jax.ShapeDtypeStruct((n, d), jnp.float32),
                   jax.ShapeDtypeStruct((n, d), jnp.float32)],
    )(a0, a1, ys, dinv, b2, fc1_t, fc1_b_half, fc1_bias)


def _tc_decoder_mlp(g, f2, b2, f3, b3, f4, b4):
    E, d = g.shape
    h2, h3 = f2.shape[1], f3.shape[1]
    EB = 2048
    assert E % EB == 0

    def body(gr, f2r, b2r, f3r, b3r, f4r, b4r, o):
        t = jnp.maximum(gr[...], 0.0)
        t = jnp.maximum(
            jnp.dot(t, f2r[...], preferred_element_type=jnp.float32)
            + b2r[...], 0.0)
        t = jnp.maximum(
            jnp.dot(t, f3r[...], preferred_element_type=jnp.float32)
            + b3r[...], 0.0)
        o[...] = jnp.dot(t, f4r[...],
                         preferred_element_type=jnp.float32) + b4r[...]

    return pl.pallas_call(
        body,
        grid=(E // EB,),
        in_specs=[pl.BlockSpec((EB, d), lambda i: (i, 0)),
                  pl.BlockSpec((d, h2), lambda i: (0, 0)),
                  pl.BlockSpec((1, h2), lambda i: (0, 0)),
                  pl.BlockSpec((h2, h3), lambda i: (0, 0)),
                  pl.BlockSpec((1, h3), lambda i: (0, 0)),
                  pl.BlockSpec((h3, 1), lambda i: (0, 0)),
                  pl.BlockSpec((1, 1), lambda i: (0, 0))],
        out_specs=pl.BlockSpec((EB, 1), lambda i: (i, 0)),
        out_shape=jax.ShapeDtypeStruct((E, 1), jnp.float32),
    )(g, f2, b2, f3, b3, f4, b4)


def kernel(x, edge_index, W1, b1, W2, b2, fc1_W, fc1_b,
           fc2_W, fc2_b, fc3_W, fc3_b, fc4_W, fc4_b):
    n, d = x.shape
    E = edge_index.shape[1]
    nchunk = -(-E // CH)
    rpw = (-(-nchunk // NW) + 7) & ~7
    epad = rpw * NW * CH
    npd = n + CH
    ei = edge_index.astype(jnp.int32)
    fill = n + (jnp.arange(epad - E, dtype=jnp.int32) % CH)
    es2d = jnp.concatenate([ei[0], fill]).reshape(epad // CH, CH)
    ed2d = jnp.concatenate([ei[1], fill]).reshape(epad // CH, CH)
    x_pad = jnp.concatenate([x, jnp.zeros((npd - n, d), jnp.float32)])

    zerosd = jnp.zeros((npd, d), jnp.float32)
    ones = jnp.ones((CH, d), jnp.float32)

    deg = _sc_degree(ed2d, ones, zerosd, npd, d)
    xs, dinv = _tc_prescale(deg[0], deg[1], x_pad, n)
    agg1 = _sc_aggregate(xs, es2d, ed2d, zerosd, npd, d)
    ys = _tc_mid(agg1[0], agg1[1], xs, dinv, W1, b1.reshape(1, -1), W2, n)
    agg2 = _sc_aggregate(ys, es2d, ed2d, zerosd, npd, d)
    a0, a1 = _tc_decoder_pre(agg2[0], agg2[1], ys, dinv, b2.reshape(1, -1),
                             fc1_W[:d], fc1_W[d:], fc1_b.reshape(1, -1), n)
    g = _sc_edge_gather(a0, a1, es2d, ed2d, d).reshape(epad, d)
    out = _tc_decoder_mlp(g, fc2_W, fc2_b.reshape(1, -1),
                          fc3_W, fc3_b.reshape(1, -1),
                          fc4_W, fc4_b.reshape(1, -1))
    return jnp.squeeze(out, axis=-1)[:E]
